# baseline (speedup 1.0000x reference)
import functools

import jax
import jax.numpy as jnp
from jax import lax
from jax.experimental import pallas as pl
from jax.experimental.pallas import tpu as pltpu
from jax.experimental.pallas import tpu_sc as plsc

BATCH = 16384
FIELDS = 26
DIM = 32
NUM_WORKERS = 32
ROWS_PER_W = BATCH * FIELDS // NUM_WORKERS
CHUNK = 1024
NCHUNK = ROWS_PER_W // CHUNK
NBUF = 3

_mesh = plsc.VectorSubcoreMesh(core_axis_name="c", subcore_axis_name="s")


@functools.partial(
    pl.kernel,
    mesh=_mesh,
    out_type=jax.ShapeDtypeStruct((BATCH * FIELDS, DIM), jnp.float32),
    scratch_types=[
        pltpu.VMEM((NCHUNK, CHUNK), jnp.int32),
        pltpu.VMEM((NBUF, CHUNK, DIM), jnp.float32),
        pltpu.SemaphoreType.DMA((NBUF,)),
        pltpu.SemaphoreType.DMA((NBUF,)),
    ],
    compiler_params=pltpu.CompilerParams(
        use_tc_tiling_on_sc=False, needs_layout_passes=False
    ),
)
def _gather_flat(x_hbm, table_hbm, out_hbm, idx_v, rows_v, gsem, ssem):
    wid = lax.axis_index("s") * 2 + lax.axis_index("c")
    r0 = wid * ROWS_PER_W

    pltpu.sync_copy(x_hbm.at[wid], idx_v)

    def start_gather(k):
        b = k % NBUF
        return pltpu.async_copy(
            table_hbm.at[idx_v.at[k]], rows_v.at[b], gsem.at[b]
        )

    gathers = [None] * NBUF
    stores = [None] * NBUF
    for k in range(NBUF - 1):
        gathers[k % NBUF] = start_gather(k)
    for k in range(NCHUNK):
        b = k % NBUF
        nk = k + NBUF - 1
        if nk < NCHUNK:
            nb = nk % NBUF
            if stores[nb] is not None:
                stores[nb].wait()
                stores[nb] = None
            gathers[nb] = start_gather(nk)
        gathers[b].wait()
        if stores[b] is not None:
            stores[b].wait()
        stores[b] = pltpu.async_copy(
            rows_v.at[b],
            out_hbm.at[pl.ds(r0 + k * CHUNK, CHUNK)],
            ssem.at[b],
        )
    for s in stores:
        if s is not None:
            s.wait()


def kernel(x, table):
    xw = x.reshape(NUM_WORKERS, NCHUNK, CHUNK)
    out = _gather_flat(xw, table)
    return out.reshape(BATCH, FIELDS, DIM)
